# baseline (speedup 1.0000x reference)
import jax
import jax.numpy as jnp
from jax import lax
from jax.experimental import pallas as pl
from jax.experimental.pallas import tpu as pltpu
from jax.experimental.pallas import tpu_sc as plsc

VOCAB = 100000
EMBED = 64
BATCH = 1024
SEQ = 50
SUB = 8
N_TOK = BATCH * SEQ
NUM_WORKERS = 32
TOK_PER_WORKER = N_TOK // NUM_WORKERS
CHUNK = 80
IDS_PER_CHUNK = CHUNK * SUB
NUM_CHUNKS = TOK_PER_WORKER // CHUNK
LANES = 16


def _sc_body(ids_hbm, w_hbm, out_hbm, idx_v, idxt_v, rows_v, out_v, scale_v,
             sem_g0, sem_g1, sem_i, sem_o0, sem_o1):
    sem_g = (sem_g0, sem_g1)
    sem_o = (sem_o0, sem_o1)
    num_cores = 2
    wid = lax.axis_index("s") * num_cores + lax.axis_index("c")
    blk0 = wid * NUM_CHUNKS

    pending_i = {}
    pending_g = {}
    pending_o = {}

    def fire_idx(ci):
        b = ci % 3
        off = (blk0 + ci) * IDS_PER_CHUNK
        pending_i[ci] = pltpu.async_copy(
            ids_hbm.at[pl.ds(off, IDS_PER_CHUNK)], idx_v.at[b], sem_i
        )

    iot = lax.iota(jnp.int32, LANES)
    dnums = lax.GatherDimensionNumbers(
        offset_dims=(), collapsed_slice_dims=(0,), start_index_map=(0,)
    )

    def vperm(x, p):
        return lax.gather(
            x, jnp.reshape(p, (LANES, 1)), dnums, (1,),
            mode=lax.GatherScatterMode.PROMISE_IN_BOUNDS,
        )

    s_perms = [s + SUB * (iot & 1) for s in range(SUB)]
    q_masks = [(iot >> 1) == q for q in range(SUB)]

    def transpose_idx(ci):
        bi = ci % 3
        b = ci & 1
        def m_body(m, inner):
            srcs = [
                idx_v[bi, pl.ds((m * SUB + q) * LANES, LANES)]
                for q in range(SUB)
            ]
            o = m * LANES
            for s in range(SUB):
                out = vperm(srcs[0], s_perms[s])
                for q in range(1, SUB):
                    out = jnp.where(q_masks[q], vperm(srcs[q], s_perms[s]), out)
                idxt_v[b, pl.ds(s * CHUNK + o, LANES)] = out
            return inner

        lax.fori_loop(0, CHUNK // LANES, m_body, 0)

    def fire_gathers(ci):
        b = ci & 1
        pending_g[ci] = [
            pltpu.async_copy(
                w_hbm.at[idxt_v.at[b, pl.ds(s * CHUNK, CHUNK)]],
                rows_v.at[b, s],
                sem_g[b],
            )
            for s in range(SUB)
        ]

    def compute_scale(ci):
        b = ci & 1
        for g in range(CHUNK // LANES):
            cnt = jnp.zeros((LANES,), jnp.float32)
            for s in range(SUB):
                ids16 = idxt_v[b, pl.ds(s * CHUNK + g * LANES, LANES)]
                cnt = cnt + jnp.where(
                    ids16 != 0, jnp.float32(1.0), jnp.float32(0.0)
                )
            scale_v[b, pl.ds(g * LANES, LANES)] = 1.0 / (cnt + 1e-9)

    def compute_chunk(ci):
        b = ci & 1

        def tok_body(t, inner):
            sc = scale_v[b, pl.ds(t, LANES)][0]
            for f in range(EMBED // LANES):
                acc = rows_v[b, 0, t, pl.ds(f * LANES, LANES)]
                for s in range(1, SUB):
                    acc = acc + rows_v[b, s, t, pl.ds(f * LANES, LANES)]
                out_v[b, t, pl.ds(f * LANES, LANES)] = acc * sc
            return inner

        lax.fori_loop(0, CHUNK, tok_body, 0, unroll=2)

    def fire_out(ci):
        b = ci & 1
        tb = (blk0 + ci) * CHUNK
        pending_o[ci] = pltpu.async_copy(
            out_v.at[b], out_hbm.at[pl.ds(tb, CHUNK)], sem_o[b]
        )

    fire_idx(0)
    fire_idx(1)
    pending_i.pop(0).wait()
    transpose_idx(0)
    fire_gathers(0)
    compute_scale(0)

    for ci in range(NUM_CHUNKS):
        if ci + 2 < NUM_CHUNKS:
            fire_idx(ci + 2)
        if ci + 1 < NUM_CHUNKS:
            pending_i.pop(ci + 1).wait()
            transpose_idx(ci + 1)
            fire_gathers(ci + 1)
            compute_scale(ci + 1)
        for c in pending_g.pop(ci):
            c.wait()
        if ci - 2 in pending_o:
            pending_o.pop(ci - 2).wait()
        compute_chunk(ci)
        fire_out(ci)

    pending_o.pop(NUM_CHUNKS - 2).wait()
    pending_o.pop(NUM_CHUNKS - 1).wait()


_mesh = plsc.VectorSubcoreMesh(core_axis_name="c", subcore_axis_name="s")

_sc_call = pl.kernel(
    _sc_body,
    out_type=jax.ShapeDtypeStruct((N_TOK, EMBED), jnp.float32),
    mesh=_mesh,
    scratch_types=[
        pltpu.VMEM((3, IDS_PER_CHUNK), jnp.int32),
        pltpu.VMEM((2, IDS_PER_CHUNK), jnp.int32),
        pltpu.VMEM((2, SUB, CHUNK, EMBED), jnp.float32),
        pltpu.VMEM((2, CHUNK, EMBED), jnp.float32),
        pltpu.VMEM((2, CHUNK + LANES), jnp.float32),
        pltpu.SemaphoreType.DMA,
        pltpu.SemaphoreType.DMA,
        pltpu.SemaphoreType.DMA,
        pltpu.SemaphoreType.DMA,
        pltpu.SemaphoreType.DMA,
    ],
    compiler_params=pltpu.CompilerParams(use_tc_tiling_on_sc=False),
)


def kernel(subtokens, W):
    ids_flat = subtokens.astype(jnp.int32).reshape(-1)
    out = _sc_call(ids_flat, W)
    return out.reshape(BATCH, SEQ, EMBED)


# speedup vs baseline: 1.1610x; 1.1610x over previous
import jax
import jax.numpy as jnp
from jax import lax
from jax.experimental import pallas as pl
from jax.experimental.pallas import tpu as pltpu
from jax.experimental.pallas import tpu_sc as plsc

VOCAB = 100000
EMBED = 64
BATCH = 1024
SEQ = 50
SUB = 8
N_TOK = BATCH * SEQ
NUM_WORKERS = 32
TOK_PER_WORKER = N_TOK // NUM_WORKERS
CHUNK = 80
NUM_CHUNKS = TOK_PER_WORKER // CHUNK
LANES = 16


def _sc_body(ids_hbm, w_hbm, out_hbm, idx_v, rows_v, out_v, scale_v,
             sem_g0, sem_g1, sem_i0, sem_i1, sem_o0, sem_o1):
    sem_g = (sem_g0, sem_g1)
    sem_i = (sem_i0, sem_i1)
    sem_o = (sem_o0, sem_o1)
    num_cores = 2
    wid = lax.axis_index("s") * num_cores + lax.axis_index("c")
    blk0 = wid * NUM_CHUNKS

    pending_i = {}
    pending_g = {}
    pending_o = {}

    def fire_idx(ci):
        b = ci & 1
        off = (blk0 + ci) * (CHUNK * SUB)
        pending_i[ci] = pltpu.async_copy(
            ids_hbm.at[pl.ds(off, CHUNK * SUB)], idx_v.at[b], sem_i[b]
        )

    def fire_gathers(ci):
        b = ci & 1
        pending_g[ci] = [
            pltpu.async_copy(
                w_hbm.at[idx_v.at[b, pl.ds(s * CHUNK, CHUNK)]],
                rows_v.at[b, s],
                sem_g[b],
            )
            for s in range(SUB)
        ]

    def compute_scale(ci):
        b = ci & 1
        for g in range(CHUNK // LANES):
            cnt = jnp.zeros((LANES,), jnp.float32)
            for s in range(SUB):
                ids16 = idx_v[b, pl.ds(s * CHUNK + g * LANES, LANES)]
                cnt = cnt + jnp.where(
                    ids16 != 0, jnp.float32(1.0), jnp.float32(0.0)
                )
            scale_v[b, pl.ds(g * LANES, LANES)] = 1.0 / (cnt + 1e-9)

    def compute_chunk(ci):
        b = ci & 1

        def tok_body(t, inner):
            sc = scale_v[b, pl.ds(t, LANES)][0]
            for f in range(EMBED // LANES):
                acc = rows_v[b, 0, t, pl.ds(f * LANES, LANES)]
                for s in range(1, SUB):
                    acc = acc + rows_v[b, s, t, pl.ds(f * LANES, LANES)]
                out_v[b, t, pl.ds(f * LANES, LANES)] = acc * sc
            return inner

        lax.fori_loop(0, CHUNK, tok_body, 0)

    def fire_out(ci):
        b = ci & 1
        tb = (blk0 + ci) * CHUNK
        pending_o[ci] = pltpu.async_copy(
            out_v.at[b], out_hbm.at[pl.ds(tb, CHUNK)], sem_o[b]
        )

    fire_idx(0)
    pending_i.pop(0).wait()
    fire_gathers(0)
    compute_scale(0)
    fire_idx(1)

    for ci in range(NUM_CHUNKS):
        if ci + 1 < NUM_CHUNKS:
            pending_i.pop(ci + 1).wait()
            fire_gathers(ci + 1)
            compute_scale(ci + 1)
        for c in pending_g.pop(ci):
            c.wait()
        if ci + 2 < NUM_CHUNKS:
            fire_idx(ci + 2)
        if ci - 2 in pending_o:
            pending_o.pop(ci - 2).wait()
        compute_chunk(ci)
        fire_out(ci)

    pending_o.pop(NUM_CHUNKS - 2).wait()
    pending_o.pop(NUM_CHUNKS - 1).wait()


_mesh = plsc.VectorSubcoreMesh(core_axis_name="c", subcore_axis_name="s")

_sc_call = pl.kernel(
    _sc_body,
    out_type=jax.ShapeDtypeStruct((N_TOK, EMBED), jnp.float32),
    mesh=_mesh,
    scratch_types=[
        pltpu.VMEM((2, SUB * CHUNK), jnp.int32),
        pltpu.VMEM((2, SUB, CHUNK, EMBED), jnp.float32),
        pltpu.VMEM((2, CHUNK, EMBED), jnp.float32),
        pltpu.VMEM((2, CHUNK + LANES), jnp.float32),
        pltpu.SemaphoreType.DMA,
        pltpu.SemaphoreType.DMA,
        pltpu.SemaphoreType.DMA,
        pltpu.SemaphoreType.DMA,
        pltpu.SemaphoreType.DMA,
        pltpu.SemaphoreType.DMA,
    ],
    compiler_params=pltpu.CompilerParams(use_tc_tiling_on_sc=False),
)


def kernel(subtokens, W):
    ids = subtokens.reshape(N_TOK, SUB).astype(jnp.int32)
    ids_blocked = (
        ids.reshape(NUM_WORKERS, NUM_CHUNKS, CHUNK, SUB)
        .transpose(0, 1, 3, 2)
        .reshape(-1)
    )
    out = _sc_call(ids_blocked, W)
    return out.reshape(BATCH, SEQ, EMBED)
